# baseline (speedup 1.0000x reference)
import jax
import jax.numpy as jnp
from jax import lax
from jax.experimental import pallas as pl
from jax.experimental.pallas import tpu as pltpu
from jax.experimental.pallas import tpu_sc as plsc

_BATCH = 128
_SEQ = 512
_HID = 768
_PAD = 1
_EPS = 1e-12
_L = 16
_NW = 32
_ROWS_PER_W = _BATCH // _NW
_CH = 64
_NCH = _SEQ // _CH
_HV = _HID // _L


def _rsqrt_nr(x):
    i = lax.bitcast_convert_type(x, jnp.int32)
    i = jnp.int32(0x5F3759DF) - lax.shift_right_logical(i, 1)
    y = lax.bitcast_convert_type(i, jnp.float32)
    for _ in range(3):
        y = y * (1.5 - 0.5 * x * y * y)
    return y


def _body(ids_hbm, wemb_hbm, pemb_hbm, gb_hbm, out_hbm,
          ids_v, pos_v, wbuf, pbuf, gb_v, sem_w, sem_p):
    cid = lax.axis_index("c")
    sid = lax.axis_index("s")
    wid = sid * 2 + cid

    pltpu.sync_copy(gb_hbm, gb_v)

    def row_body(r, _):
        row = wid * _ROWS_PER_W + r
        pltpu.sync_copy(ids_hbm.at[row], ids_v)

        def pos_body(i, carry):
            seg = ids_v[pl.ds(i * _L, _L)]
            m = seg != _PAD
            mi = jnp.where(m, jnp.int32(1), jnp.int32(0))
            cs = plsc.cumsum(mi)
            pos_v[pl.ds(i * _L, _L)] = jnp.where(m, cs + (carry + 1),
                                                 jnp.int32(_PAD))
            return carry + jnp.sum(mi)

        lax.fori_loop(0, _SEQ // _L, pos_body, jnp.int32(0))

        def chunk_body(c, _):
            idx_w = ids_v.at[pl.ds(c * _CH, _CH)]
            idx_p = pos_v.at[pl.ds(c * _CH, _CH)]
            cw = pltpu.async_copy(wemb_hbm.at[idx_w], wbuf, sem_w)
            cp = pltpu.async_copy(pemb_hbm.at[idx_p], pbuf, sem_p)
            cw.wait()
            cp.wait()

            def tok_body(t, _):
                def acc_body(i, accs):
                    a1, a2 = accs
                    sl = pl.ds(i * _L, _L)
                    x = wbuf[t, sl] + pbuf[t, sl]
                    wbuf[t, sl] = x
                    return (a1 + x, a2 + x * x)

                zero = jnp.zeros((_L,), jnp.float32)
                a1, a2 = lax.fori_loop(0, _HV, acc_body, (zero, zero))
                mean = jnp.sum(a1) * (1.0 / _HID)
                ex2 = jnp.sum(a2) * (1.0 / _HID)
                var = ex2 - mean * mean
                rstd_v = _rsqrt_nr(jnp.full((_L,), var + _EPS, jnp.float32))
                mean_v = jnp.full((_L,), mean, jnp.float32)

                def norm_body(i, _):
                    sl = pl.ds(i * _L, _L)
                    x = wbuf[t, sl]
                    wbuf[t, sl] = ((x - mean_v) * rstd_v * gb_v[0, sl]
                                   + gb_v[1, sl])
                    return 0

                lax.fori_loop(0, _HV, norm_body, 0)
                return 0

            lax.fori_loop(0, _CH, tok_body, 0)
            pltpu.sync_copy(wbuf,
                            out_hbm.at[pl.ds(row * _SEQ + c * _CH, _CH)])
            return 0

        lax.fori_loop(0, _NCH, chunk_body, 0)
        return 0

    lax.fori_loop(0, _ROWS_PER_W, row_body, 0)


@jax.jit
def kernel(input_ids, word_emb, pos_emb, gamma, beta):
    ids = input_ids.astype(jnp.int32)
    gb = jnp.stack([gamma, beta]).astype(jnp.float32)
    mesh = plsc.VectorSubcoreMesh(core_axis_name="c", subcore_axis_name="s")
    out = pl.kernel(
        _body,
        out_type=jax.ShapeDtypeStruct((_BATCH * _SEQ, _HID), jnp.float32),
        mesh=mesh,
        compiler_params=pltpu.CompilerParams(needs_layout_passes=False),
        scratch_types=[
            pltpu.VMEM((_SEQ,), jnp.int32),
            pltpu.VMEM((_SEQ,), jnp.int32),
            pltpu.VMEM((_CH, _HID), jnp.float32),
            pltpu.VMEM((_CH, _HID), jnp.float32),
            pltpu.VMEM((2, _HID), jnp.float32),
            pltpu.SemaphoreType.DMA,
            pltpu.SemaphoreType.DMA,
        ],
    )(ids, word_emb, pos_emb, gb)
    return out.reshape(_BATCH, _SEQ, _HID)


# speedup vs baseline: 1.1236x; 1.1236x over previous
import jax
import jax.numpy as jnp
from jax import lax
from jax.experimental import pallas as pl
from jax.experimental.pallas import tpu as pltpu
from jax.experimental.pallas import tpu_sc as plsc

_BATCH = 128
_SEQ = 512
_HID = 768
_PAD = 1
_EPS = 1e-12
_L = 16
_NW = 32
_TOK_PER_W = _BATCH * _SEQ // _NW
_ROWS_PER_W = _BATCH // _NW
_CH = 32
_NCH = _TOK_PER_W // _CH
_HV = _HID // _L
_UNROLL = 4


def _rsqrt_nr(x):
    i = lax.bitcast_convert_type(x, jnp.int32)
    i = jnp.int32(0x5F3759DF) - lax.shift_right_logical(i, 1)
    y = lax.bitcast_convert_type(i, jnp.float32)
    for _ in range(3):
        y = y * (1.5 - 0.5 * x * y * y)
    return y


def _body(ids_hbm, wemb_hbm, pemb_hbm, gb_hbm, out_hbm,
          ids_v, pos_v, wbuf0, wbuf1, pbuf0, pbuf1, gb_v,
          sem_w0, sem_w1, sem_p0, sem_p1, sem_o0, sem_o1):
    cid = lax.axis_index("c")
    sid = lax.axis_index("s")
    wid = sid * 2 + cid
    base = wid * _TOK_PER_W

    wbufs = (wbuf0, wbuf1)
    pbufs = (pbuf0, pbuf1)
    sem_w = (sem_w0, sem_w1)
    sem_p = (sem_p0, sem_p1)
    sem_o = (sem_o0, sem_o1)

    pltpu.sync_copy(gb_hbm, gb_v)
    pltpu.sync_copy(ids_hbm.at[pl.ds(base, _TOK_PER_W)], ids_v)

    def issue_w(c, buf, sem):
        idx = ids_v.at[pl.ds(c * _CH, _CH)]
        pltpu.async_copy(wemb_hbm.at[idx], buf, sem)

    def issue_p(c, buf, sem):
        idx = pos_v.at[pl.ds(c * _CH, _CH)]
        pltpu.async_copy(pemb_hbm.at[idx], buf, sem)

    def wait_into(buf, sem):
        pltpu.make_async_copy(out_hbm.at[pl.ds(0, _CH)], buf, sem).wait()

    issue_w(0, wbuf0, sem_w0)

    def pos_row(r, _):
        def pos_body(i, carry):
            o = r * _SEQ + i * _L
            seg = ids_v[pl.ds(o, _L)]
            m = seg != _PAD
            mi = jnp.where(m, jnp.int32(1), jnp.int32(0))
            cs = plsc.cumsum(mi)
            pos_v[pl.ds(o, _L)] = jnp.where(m, cs + (carry + 1),
                                            jnp.int32(_PAD))
            return carry + jnp.sum(mi)

        lax.fori_loop(0, _SEQ // _L, pos_body, jnp.int32(0))
        return 0

    lax.fori_loop(0, _ROWS_PER_W, pos_row, 0)

    def compute_chunk(buf, pb):
        def tok_body(t, _):
            def acc_body(i, accs):
                a1, a2 = accs
                for u in range(_UNROLL):
                    sl = pl.ds((i * _UNROLL + u) * _L, _L)
                    x = buf[t, sl] + pb[t, sl]
                    buf[t, sl] = x
                    a1 = a1 + x
                    a2 = a2 + x * x
                return (a1, a2)

            zero = jnp.zeros((_L,), jnp.float32)
            a1, a2 = lax.fori_loop(0, _HV // _UNROLL, acc_body, (zero, zero))
            mean = jnp.sum(a1) * (1.0 / _HID)
            ex2 = jnp.sum(a2) * (1.0 / _HID)
            var = ex2 - mean * mean
            rstd_v = _rsqrt_nr(jnp.full((_L,), var + _EPS, jnp.float32))
            mean_v = jnp.full((_L,), mean, jnp.float32)

            def norm_body(i, _):
                for u in range(_UNROLL):
                    sl = pl.ds((i * _UNROLL + u) * _L, _L)
                    x = buf[t, sl]
                    buf[t, sl] = ((x - mean_v) * rstd_v * gb_v[0, sl]
                                  + gb_v[1, sl])
                return 0

            lax.fori_loop(0, _HV // _UNROLL, norm_body, 0)
            return 0

        lax.fori_loop(0, _CH, tok_body, 0)

    issue_p(0, pbuf0, sem_p0)

    def pair_body(i, _):
        for b in range(2):
            c = 2 * i + b
            wait_into(wbufs[b], sem_w[b])
            wait_into(pbufs[b], sem_p[b])

            @pl.when(c > 0)
            def _():
                pltpu.make_async_copy(
                    wbufs[1 - b], out_hbm.at[pl.ds(0, _CH)],
                    sem_o[1 - b]).wait()

            @pl.when(c < _NCH - 1)
            def _():
                issue_w(c + 1, wbufs[1 - b], sem_w[1 - b])
                issue_p(c + 1, pbufs[1 - b], sem_p[1 - b])

            compute_chunk(wbufs[b], pbufs[b])
            pltpu.async_copy(wbufs[b],
                             out_hbm.at[pl.ds(base + c * _CH, _CH)],
                             sem_o[b])
        return 0

    lax.fori_loop(0, _NCH // 2, pair_body, 0)

    pltpu.make_async_copy(wbufs[(_NCH - 1) % 2], out_hbm.at[pl.ds(0, _CH)],
                          sem_o[(_NCH - 1) % 2]).wait()


@jax.jit
def kernel(input_ids, word_emb, pos_emb, gamma, beta):
    ids = input_ids.astype(jnp.int32).reshape(_BATCH * _SEQ)
    gb = jnp.stack([gamma, beta]).astype(jnp.float32)
    mesh = plsc.VectorSubcoreMesh(core_axis_name="c", subcore_axis_name="s")
    out = pl.kernel(
        _body,
        out_type=jax.ShapeDtypeStruct((_BATCH * _SEQ, _HID), jnp.float32),
        mesh=mesh,
        compiler_params=pltpu.CompilerParams(needs_layout_passes=False),
        scratch_types=[
            pltpu.VMEM((_TOK_PER_W,), jnp.int32),
            pltpu.VMEM((_TOK_PER_W,), jnp.int32),
            pltpu.VMEM((_CH, _HID), jnp.float32),
            pltpu.VMEM((_CH, _HID), jnp.float32),
            pltpu.VMEM((_CH, _HID), jnp.float32),
            pltpu.VMEM((_CH, _HID), jnp.float32),
            pltpu.VMEM((2, _HID), jnp.float32),
            pltpu.SemaphoreType.DMA,
            pltpu.SemaphoreType.DMA,
            pltpu.SemaphoreType.DMA,
            pltpu.SemaphoreType.DMA,
            pltpu.SemaphoreType.DMA,
            pltpu.SemaphoreType.DMA,
        ],
    )(ids, word_emb, pos_emb, gb)
    return out.reshape(_BATCH, _SEQ, _HID)


# speedup vs baseline: 2.6432x; 2.3524x over previous
import jax
import jax.numpy as jnp
from jax import lax
from jax.experimental import pallas as pl
from jax.experimental.pallas import tpu as pltpu
from jax.experimental.pallas import tpu_sc as plsc

_BATCH = 128
_SEQ = 512
_HID = 768
_PAD = 1
_EPS = 1e-12
_L = 16
_NW = 32
_TOK_PER_W = _BATCH * _SEQ // _NW
_ROWS_PER_W = _BATCH // _NW
_CH = 32
_NCH = _TOK_PER_W // _CH
_HV = _HID // _L
_UNROLL = 4


def _rsqrt_nr(x):
    i = lax.bitcast_convert_type(x, jnp.int32)
    i = jnp.int32(0x5F3759DF) - lax.shift_right_logical(i, 1)
    y = lax.bitcast_convert_type(i, jnp.float32)
    for _ in range(3):
        y = y * (1.5 - 0.5 * x * y * y)
    return y


def _body(ids_hbm, wemb_hbm, pemb_hbm, gb_hbm, out_hbm,
          ids_v, pos_v, wbuf0, wbuf1, pbuf0, pbuf1, gb_v,
          sem_w0, sem_w1, sem_p0, sem_p1, sem_o0, sem_o1):
    cid = lax.axis_index("c")
    sid = lax.axis_index("s")
    wid = sid * 2 + cid
    base = wid * _TOK_PER_W

    wbufs = (wbuf0, wbuf1)
    pbufs = (pbuf0, pbuf1)
    sem_w = (sem_w0, sem_w1)
    sem_p = (sem_p0, sem_p1)
    sem_o = (sem_o0, sem_o1)

    pltpu.sync_copy(gb_hbm, gb_v)
    pltpu.sync_copy(ids_hbm.at[pl.ds(base, _TOK_PER_W)], ids_v)

    def issue_w(c, buf, sem):
        idx = ids_v.at[pl.ds(c * _CH, _CH)]
        pltpu.async_copy(wemb_hbm.at[idx], buf, sem)

    def issue_p(c, buf, sem):
        idx = pos_v.at[pl.ds(c * _CH, _CH)]
        pltpu.async_copy(pemb_hbm.at[idx], buf, sem)

    def wait_into(buf, sem):
        pltpu.make_async_copy(out_hbm.at[pl.ds(0, _CH)], buf, sem).wait()

    issue_w(0, wbuf0, sem_w0)

    def pos_row(r, _):
        def pos_body(i, carry):
            o = r * _SEQ + i * _L
            seg = ids_v[pl.ds(o, _L)]
            m = seg != _PAD
            mi = jnp.where(m, jnp.int32(1), jnp.int32(0))
            cs = plsc.cumsum(mi)
            pos_v[pl.ds(o, _L)] = jnp.where(m, cs + (carry + 1),
                                            jnp.int32(_PAD))
            return carry + jnp.sum(mi)

        lax.fori_loop(0, _SEQ // _L, pos_body, jnp.int32(0))
        return 0

    lax.fori_loop(0, _ROWS_PER_W, pos_row, 0)

    def compute_chunk(buf, pb):
        @plsc.parallel_loop(0, _CH, 1)
        def tok_body(t):
            zero = jnp.zeros((_L,), jnp.float32)
            accs = [zero, zero, zero, zero]
            for g in range(_HV):
                sl = pl.ds(g * _L, _L)
                x = buf[t, sl] + pb[t, sl]
                buf[t, sl] = x
                accs[2 * (g % 2)] = accs[2 * (g % 2)] + x
                accs[2 * (g % 2) + 1] = accs[2 * (g % 2) + 1] + x * x
            mean = jnp.sum(accs[0] + accs[2]) * (1.0 / _HID)
            ex2 = jnp.sum(accs[1] + accs[3]) * (1.0 / _HID)
            var = ex2 - mean * mean
            rstd_v = _rsqrt_nr(jnp.full((_L,), var + _EPS, jnp.float32))
            mean_v = jnp.full((_L,), mean, jnp.float32)
            for g in range(_HV):
                sl = pl.ds(g * _L, _L)
                x = buf[t, sl]
                buf[t, sl] = ((x - mean_v) * rstd_v * gb_v[0, sl]
                              + gb_v[1, sl])

    issue_p(0, pbuf0, sem_p0)

    def pair_body(i, _):
        for b in range(2):
            c = 2 * i + b
            wait_into(wbufs[b], sem_w[b])
            wait_into(pbufs[b], sem_p[b])

            @pl.when(c > 0)
            def _():
                pltpu.make_async_copy(
                    wbufs[1 - b], out_hbm.at[pl.ds(0, _CH)],
                    sem_o[1 - b]).wait()

            @pl.when(c < _NCH - 1)
            def _():
                issue_w(c + 1, wbufs[1 - b], sem_w[1 - b])
                issue_p(c + 1, pbufs[1 - b], sem_p[1 - b])

            compute_chunk(wbufs[b], pbufs[b])
            pltpu.async_copy(wbufs[b],
                             out_hbm.at[pl.ds(base + c * _CH, _CH)],
                             sem_o[b])
        return 0

    lax.fori_loop(0, _NCH // 2, pair_body, 0)

    pltpu.make_async_copy(wbufs[(_NCH - 1) % 2], out_hbm.at[pl.ds(0, _CH)],
                          sem_o[(_NCH - 1) % 2]).wait()


@jax.jit
def kernel(input_ids, word_emb, pos_emb, gamma, beta):
    ids = input_ids.astype(jnp.int32).reshape(_BATCH * _SEQ)
    gb = jnp.stack([gamma, beta]).astype(jnp.float32)
    mesh = plsc.VectorSubcoreMesh(core_axis_name="c", subcore_axis_name="s")
    out = pl.kernel(
        _body,
        out_type=jax.ShapeDtypeStruct((_BATCH * _SEQ, _HID), jnp.float32),
        mesh=mesh,
        compiler_params=pltpu.CompilerParams(needs_layout_passes=False),
        scratch_types=[
            pltpu.VMEM((_TOK_PER_W,), jnp.int32),
            pltpu.VMEM((_TOK_PER_W,), jnp.int32),
            pltpu.VMEM((_CH, _HID), jnp.float32),
            pltpu.VMEM((_CH, _HID), jnp.float32),
            pltpu.VMEM((_CH, _HID), jnp.float32),
            pltpu.VMEM((_CH, _HID), jnp.float32),
            pltpu.VMEM((2, _HID), jnp.float32),
            pltpu.SemaphoreType.DMA,
            pltpu.SemaphoreType.DMA,
            pltpu.SemaphoreType.DMA,
            pltpu.SemaphoreType.DMA,
            pltpu.SemaphoreType.DMA,
            pltpu.SemaphoreType.DMA,
        ],
    )(ids, word_emb, pos_emb, gb)
    return out.reshape(_BATCH, _SEQ, _HID)
